# baseline (speedup 1.0000x reference)
import functools

import jax
import jax.numpy as jnp
from jax import lax
from jax.experimental import pallas as pl
from jax.experimental.pallas import tpu as pltpu
from jax.experimental.pallas import tpu_sc as plsc

VOCAB = 100000
EMB = 32
CTX = 20
BATCH = 1024
FEAT = CTX * EMB

_NC = 2
_NS = 16
_NW = _NC * _NS
_CHUNK = 128

_N_IDX = BATCH * CTX
_ROWS = _N_IDX // _CHUNK
_ROWS_PER_W = _ROWS // _NW


_L = 16
_BC = 256
_NBC = BATCH // _BC


def _sc_gather(table_t, idx_t):
    mesh = plsc.VectorSubcoreMesh(core_axis_name="c", subcore_axis_name="s")

    @functools.partial(
        pl.kernel,
        mesh=mesh,
        out_type=jax.ShapeDtypeStruct((CTX * EMB, BATCH), jnp.float32),
        scratch_types=[
            pltpu.VMEM((VOCAB,), jnp.float32),
            pltpu.VMEM((2, CTX, _BC), jnp.int32),
            pltpu.VMEM((2, CTX, _BC), jnp.float32),
            pltpu.SemaphoreType.DMA,
            pltpu.SemaphoreType.DMA((2,)),
            pltpu.SemaphoreType.DMA((2,)),
        ],
        compiler_params=pltpu.CompilerParams(use_tc_tiling_on_sc=False,
                                             needs_layout_passes=False),
    )
    def k(table_hbm, idx_hbm, out_hbm, feat_v, idx_v, out_v, sem, isem, osem):
        e = lax.axis_index("s") * _NC + lax.axis_index("c")
        feat_copy = pltpu.async_copy(table_hbm.at[e], feat_v, sem)

        def idx_copy(bc, s):
            return pltpu.async_copy(
                idx_hbm.at[:, pl.ds(bc * _BC, _BC)], idx_v.at[s], isem.at[s])

        ic = [None] * _NBC
        ic[0] = idx_copy(0, 0)
        ic[1] = idx_copy(1, 1)
        feat_copy.wait()

        out_copies = [None] * _NBC
        for bc in range(_NBC):
            s = bc % 2
            ic[bc].wait()
            if bc >= 2:
                for oc in out_copies[bc - 2]:
                    oc.wait()

            @plsc.parallel_loop(0, _BC // _L, unroll=4)
            def _(g):
                off = pl.multiple_of(g * _L, _L)
                for c in range(CTX):
                    vec = idx_v[s, c, pl.ds(off, _L)]
                    val = plsc.load_gather(feat_v, [vec])
                    out_v[s, c, pl.ds(off, _L)] = val

            out_copies[bc] = [
                pltpu.async_copy(
                    out_v.at[s, c],
                    out_hbm.at[c * EMB + e, pl.ds(bc * _BC, _BC)],
                    osem.at[s])
                for c in range(CTX)
            ]
            if bc + 2 < _NBC:
                ic[bc + 2] = idx_copy(bc + 2, s)

        for bc in (_NBC - 2, _NBC - 1):
            for oc in out_copies[bc]:
                oc.wait()

    return k(table_t, idx_t)


_VB = 2048
_NFULL = VOCAB // _VB
_TAIL = VOCAB - _NFULL * _VB


def _bf16_dot(w, a16):
    return lax.dot_general(
        w.astype(jnp.bfloat16), a16,
        dimension_numbers=(((1,), (0,)), ((), ())),
        preferred_element_type=jnp.float32,
    )


def _proj_body(xf_ref, b_ref, w_hbm, out_hbm,
               xf16, wbuf0, wbuf1, obuf0, obuf1, wtail, otail,
               lsem, ssem, tlsem, tssem):
    wbufs = (wbuf0, wbuf1)
    obufs = (obuf0, obuf1)

    def w_copy(j, s):
        off = pl.multiple_of(j * _VB, _VB)
        return pltpu.make_async_copy(
            w_hbm.at[pl.ds(off, _VB)], wbufs[s], lsem.at[s])

    def o_copy(j, s):
        off = pl.multiple_of(j * _VB, _VB)
        return pltpu.make_async_copy(
            obufs[s], out_hbm.at[pl.ds(off, _VB)], ssem.at[s])

    tail_load = pltpu.make_async_copy(
        w_hbm.at[pl.ds(_NFULL * _VB, _TAIL)], wtail, tlsem)
    tail_load.start()
    w_copy(0, 0).start()
    w_copy(1, 1).start()

    xf16[...] = xf_ref[...].astype(jnp.bfloat16)
    a16 = xf16[...]

    tail_load.wait()
    otail[...] = _bf16_dot(wtail[...], a16) + lax.broadcast_in_dim(b_ref[_NFULL][:_TAIL], (_TAIL, BATCH), (0,))
    tail_store = pltpu.make_async_copy(
        otail, out_hbm.at[pl.ds(_NFULL * _VB, _TAIL)], tssem)
    tail_store.start()

    def pair(p, carry):
        for s in (0, 1):
            j = 2 * p + s
            w_copy(j, s).wait()

            @pl.when(j >= 2)
            def _():
                o_copy(j - 2, s).wait()

            obufs[s][...] = _bf16_dot(wbufs[s][...], a16) + lax.broadcast_in_dim(b_ref[j], (_VB, BATCH), (0,))
            o_copy(j, s).start()

            @pl.when(j + 2 < _NFULL)
            def _():
                w_copy(j + 2, s).start()
        return carry

    lax.fori_loop(0, _NFULL // 2, pair, 0)

    o_copy(_NFULL - 2, 0).wait()
    o_copy(_NFULL - 1, 1).wait()
    tail_store.wait()


def _tc_project(xf, W, b3d):
    return pl.pallas_call(
        _proj_body,
        in_specs=[
            pl.BlockSpec(memory_space=pltpu.VMEM),
            pl.BlockSpec(memory_space=pltpu.VMEM),
            pl.BlockSpec(memory_space=pltpu.HBM),
        ],
        out_specs=pl.BlockSpec(memory_space=pltpu.HBM),
        out_shape=jax.ShapeDtypeStruct((VOCAB, BATCH), jnp.float32),
        scratch_shapes=[
            pltpu.VMEM((FEAT, BATCH), jnp.bfloat16),
            pltpu.VMEM((_VB, FEAT), jnp.float32),
            pltpu.VMEM((_VB, FEAT), jnp.float32),
            pltpu.VMEM((_VB, BATCH), jnp.float32),
            pltpu.VMEM((_VB, BATCH), jnp.float32),
            pltpu.VMEM((_TAIL, FEAT), jnp.float32),
            pltpu.VMEM((_TAIL, BATCH), jnp.float32),
            pltpu.SemaphoreType.DMA((2,)),
            pltpu.SemaphoreType.DMA((2,)),
            pltpu.SemaphoreType.DMA,
            pltpu.SemaphoreType.DMA,
        ],
        compiler_params=pltpu.CompilerParams(
            vmem_limit_bytes=63 * 1024 * 1024,
        ),
    )(xf, b3d, W)


def kernel(x, emb_table, W, b):
    table_t = emb_table.T
    idx_t = x.T.astype(jnp.int32)
    a = _sc_gather(table_t, idx_t)
    b3d = jnp.pad(b, (0, (_NFULL + 1) * _VB - VOCAB)).reshape(_NFULL + 1, _VB)
    logits_t = _tc_project(a, W, b3d)
    return logits_t.T


# speedup vs baseline: 1.0047x; 1.0047x over previous
import functools

import jax
import jax.numpy as jnp
from jax import lax
from jax.experimental import pallas as pl
from jax.experimental.pallas import tpu as pltpu
from jax.experimental.pallas import tpu_sc as plsc

VOCAB = 100000
EMB = 32
CTX = 20
BATCH = 1024
FEAT = CTX * EMB

_NC = 2
_NS = 16
_NW = _NC * _NS
_CHUNK = 128

_N_IDX = BATCH * CTX
_ROWS = _N_IDX // _CHUNK
_ROWS_PER_W = _ROWS // _NW


_L = 16
_BC = 256
_NBC = BATCH // _BC


def _sc_gather(table_t, idx_t):
    mesh = plsc.VectorSubcoreMesh(core_axis_name="c", subcore_axis_name="s")

    @functools.partial(
        pl.kernel,
        mesh=mesh,
        out_type=jax.ShapeDtypeStruct((CTX * EMB, BATCH), jnp.float32),
        scratch_types=[
            pltpu.VMEM((VOCAB,), jnp.float32),
            pltpu.VMEM((CTX, _BC), jnp.int32),
            pltpu.VMEM((CTX, BATCH), jnp.float32),
            pltpu.SemaphoreType.DMA,
            pltpu.SemaphoreType.DMA,
        ],
        compiler_params=pltpu.CompilerParams(use_tc_tiling_on_sc=False,
                                             needs_layout_passes=False),
    )
    def k(table_hbm, idx_hbm, out_hbm, feat_v, idx_v, out_v, sem, osem):
        e = lax.axis_index("s") * _NC + lax.axis_index("c")
        pltpu.async_copy(table_hbm.at[e], feat_v, sem).wait()
        for bc in range(_NBC):
            b0 = bc * _BC
            pltpu.sync_copy(idx_hbm.at[:, pl.ds(b0, _BC)], idx_v)

            @plsc.parallel_loop(0, _BC // _L, unroll=2)
            def _(g):
                off = pl.multiple_of(g * _L, _L)
                for c in range(CTX):
                    vec = idx_v[c, pl.ds(off, _L)]
                    val = plsc.load_gather(feat_v, [vec])
                    out_v[c, pl.ds(b0 + off, _L)] = val

        out_copies = [
            pltpu.async_copy(out_v.at[c], out_hbm.at[c * EMB + e], osem)
            for c in range(CTX)
        ]
        for oc in out_copies:
            oc.wait()

    return k(table_t, idx_t)


_VB = 2048
_NFULL = VOCAB // _VB
_TAIL = VOCAB - _NFULL * _VB


def _bf16_dot(w, a16):
    return lax.dot_general(
        w.astype(jnp.bfloat16), a16,
        dimension_numbers=(((1,), (0,)), ((), ())),
        preferred_element_type=jnp.float32,
    )


def _proj_body(xf_ref, b_ref, w_hbm, out_hbm,
               xf16, wbuf0, wbuf1, obuf0, obuf1, wtail, otail,
               lsem, ssem, tlsem, tssem):
    wbufs = (wbuf0, wbuf1)
    obufs = (obuf0, obuf1)

    def w_copy(j, s):
        off = pl.multiple_of(j * _VB, _VB)
        return pltpu.make_async_copy(
            w_hbm.at[pl.ds(off, _VB)], wbufs[s], lsem.at[s])

    def o_copy(j, s):
        off = pl.multiple_of(j * _VB, _VB)
        return pltpu.make_async_copy(
            obufs[s], out_hbm.at[pl.ds(off, _VB)], ssem.at[s])

    tail_load = pltpu.make_async_copy(
        w_hbm.at[pl.ds(_NFULL * _VB, _TAIL)], wtail, tlsem)
    tail_load.start()
    w_copy(0, 0).start()
    w_copy(1, 1).start()

    xf16[...] = xf_ref[...].astype(jnp.bfloat16)
    a16 = xf16[...]

    tail_load.wait()
    otail[...] = _bf16_dot(wtail[...], a16) + lax.broadcast_in_dim(b_ref[_NFULL][:_TAIL], (_TAIL, BATCH), (0,))
    tail_store = pltpu.make_async_copy(
        otail, out_hbm.at[pl.ds(_NFULL * _VB, _TAIL)], tssem)
    tail_store.start()

    def pair(p, carry):
        for s in (0, 1):
            j = 2 * p + s
            w_copy(j, s).wait()

            @pl.when(j >= 2)
            def _():
                o_copy(j - 2, s).wait()

            obufs[s][...] = _bf16_dot(wbufs[s][...], a16) + lax.broadcast_in_dim(b_ref[j], (_VB, BATCH), (0,))
            o_copy(j, s).start()

            @pl.when(j + 2 < _NFULL)
            def _():
                w_copy(j + 2, s).start()
        return carry

    lax.fori_loop(0, _NFULL // 2, pair, 0)

    o_copy(_NFULL - 2, 0).wait()
    o_copy(_NFULL - 1, 1).wait()
    tail_store.wait()


def _tc_project(xf, W, b3d):
    return pl.pallas_call(
        _proj_body,
        in_specs=[
            pl.BlockSpec(memory_space=pltpu.VMEM),
            pl.BlockSpec(memory_space=pltpu.VMEM),
            pl.BlockSpec(memory_space=pltpu.HBM),
        ],
        out_specs=pl.BlockSpec(memory_space=pltpu.HBM),
        out_shape=jax.ShapeDtypeStruct((VOCAB, BATCH), jnp.float32),
        scratch_shapes=[
            pltpu.VMEM((FEAT, BATCH), jnp.bfloat16),
            pltpu.VMEM((_VB, FEAT), jnp.float32),
            pltpu.VMEM((_VB, FEAT), jnp.float32),
            pltpu.VMEM((_VB, BATCH), jnp.float32),
            pltpu.VMEM((_VB, BATCH), jnp.float32),
            pltpu.VMEM((_TAIL, FEAT), jnp.float32),
            pltpu.VMEM((_TAIL, BATCH), jnp.float32),
            pltpu.SemaphoreType.DMA((2,)),
            pltpu.SemaphoreType.DMA((2,)),
            pltpu.SemaphoreType.DMA,
            pltpu.SemaphoreType.DMA,
        ],
        compiler_params=pltpu.CompilerParams(
            vmem_limit_bytes=63 * 1024 * 1024,
        ),
    )(xf, b3d, W)


def kernel(x, emb_table, W, b):
    table_t = emb_table.T
    idx_t = x.T.astype(jnp.int32)
    a = _sc_gather(table_t, idx_t)
    b3d = jnp.pad(b, (0, (_NFULL + 1) * _VB - VOCAB)).reshape(_NFULL + 1, _VB)
    logits_t = _tc_project(a, W, b3d)
    return logits_t.T
